# baseline (speedup 1.0000x reference)
import functools

import jax
import jax.numpy as jnp
from jax.experimental import pallas as pl


def _gcn_block(x_ref, w1_ref, b1_ref, w2_ref, b2_ref, o_ref):
    x = x_ref[...]
    n = x.shape[1]
    m = jnp.sum(x, axis=1) * (1.0 / n)
    h = jnp.dot(m, w1_ref[...], preferred_element_type=jnp.float32)
    h = h + b1_ref[...][None, :]
    y = jnp.dot(h, w2_ref[...], preferred_element_type=jnp.float32)
    y = y + b2_ref[...][None, :]
    o_ref[...] = jnp.broadcast_to(
        y[:, None, :], (x.shape[0], n, y.shape[1])
    )


@functools.partial(jax.jit, static_argnames=())
def kernel(user_batch, W1, b1, W2, b2):
    B, N, d_in = user_batch.shape
    d_hid = W1.shape[1]
    d_out = W2.shape[1]
    Gb = 128

    return pl.pallas_call(
        _gcn_block,
        grid=(B // Gb,),
        in_specs=[
            pl.BlockSpec((Gb, N, d_in), lambda i: (i, 0, 0)),
            pl.BlockSpec((d_in, d_hid), lambda i: (0, 0)),
            pl.BlockSpec((d_hid,), lambda i: (0,)),
            pl.BlockSpec((d_hid, d_out), lambda i: (0, 0)),
            pl.BlockSpec((d_out,), lambda i: (0,)),
        ],
        out_specs=pl.BlockSpec((Gb, N, d_out), lambda i: (i, 0, 0)),
        out_shape=jax.ShapeDtypeStruct((B, N, d_out), user_batch.dtype),
    )(user_batch, W1, b1, W2, b2)


# speedup vs baseline: 1.2560x; 1.2560x over previous
import functools

import jax
import jax.numpy as jnp
from jax.experimental import pallas as pl


def _gcn_block(x_ref, w1_ref, b1_ref, w2_ref, b2_ref, o_ref):
    x = x_ref[...]
    n = x.shape[1]
    m = jnp.sum(x, axis=1) * (1.0 / n)
    h = jnp.dot(m, w1_ref[...], preferred_element_type=jnp.float32)
    h = h + b1_ref[...][None, :]
    y = jnp.dot(h, w2_ref[...], preferred_element_type=jnp.float32)
    y = y + b2_ref[...][None, :]
    o_ref[...] = jnp.broadcast_to(
        y[:, None, :], (x.shape[0], n, y.shape[1])
    )


@functools.partial(jax.jit, static_argnames=())
def kernel(user_batch, W1, b1, W2, b2):
    B, N, d_in = user_batch.shape
    d_hid = W1.shape[1]
    d_out = W2.shape[1]
    Gb = 64

    return pl.pallas_call(
        _gcn_block,
        grid=(B // Gb,),
        in_specs=[
            pl.BlockSpec((Gb, N, d_in), lambda i: (i, 0, 0)),
            pl.BlockSpec((d_in, d_hid), lambda i: (0, 0)),
            pl.BlockSpec((d_hid,), lambda i: (0,)),
            pl.BlockSpec((d_hid, d_out), lambda i: (0, 0)),
            pl.BlockSpec((d_out,), lambda i: (0,)),
        ],
        out_specs=pl.BlockSpec((Gb, N, d_out), lambda i: (i, 0, 0)),
        out_shape=jax.ShapeDtypeStruct((B, N, d_out), user_batch.dtype),
    )(user_batch, W1, b1, W2, b2)
